# baseline (speedup 1.0000x reference)
import functools

import jax
import jax.numpy as jnp
from jax.experimental import pallas as pl
from jax.experimental.pallas import tpu as pltpu

_BM = 128


def _fused_kernel(x_ref, w_ref, b_ref, a0_ref, a1_ref, a_hbm, o_ref,
                  y_ref, abuf2, sem2, abuf3, sem3):
    n = x_ref.shape[0]
    q = n // 4
    nblk = n // _BM
    i = pl.program_id(0)

    @pl.when(i == 0)
    def _first():
        y_ref[...] = jnp.dot(
            x_ref[...], w_ref[...], preferred_element_type=jnp.float32
        ).astype(jnp.bfloat16)

    def _copy2(blk):
        slot = jax.lax.rem(blk, 2)
        return pltpu.make_async_copy(
            a_hbm.at[pl.ds(blk * _BM, _BM), pl.ds(2 * q, q)],
            abuf2.at[slot],
            sem2.at[slot],
        )

    def _copy3(blk):
        slot = jax.lax.rem(blk, 2)
        return pltpu.make_async_copy(
            a_hbm.at[pl.ds(blk * _BM, _BM), pl.ds(3 * q, q)],
            abuf3.at[slot],
            sem3.at[slot],
        )

    @pl.when(i == 0)
    def _prologue():
        _copy2(0).start()
        _copy3(0).start()

    @pl.when(i + 1 < nblk)
    def _prefetch():
        _copy2(i + 1).start()
        _copy3(i + 1).start()

    _copy2(i).wait()
    _copy3(i).wait()
    slot = jax.lax.rem(i, 2)
    acc = b_ref[...].astype(jnp.float32)
    acc += jnp.dot(a0_ref[...].astype(jnp.bfloat16), y_ref[0:q, :],
                   preferred_element_type=jnp.float32)
    acc += jnp.dot(a1_ref[...].astype(jnp.bfloat16), y_ref[q:2 * q, :],
                   preferred_element_type=jnp.float32)
    acc += jnp.dot(abuf2[slot].astype(jnp.bfloat16), y_ref[2 * q:3 * q, :],
                   preferred_element_type=jnp.float32)
    acc += jnp.dot(abuf3[slot].astype(jnp.bfloat16), y_ref[3 * q:n, :],
                   preferred_element_type=jnp.float32)
    o_ref[...] = acc


@jax.jit
def kernel(X, A, W, b):
    n, d_in = X.shape
    d_out = W.shape[1]

    b2 = b.reshape(1, d_out)
    q = n // 4
    out = pl.pallas_call(
        _fused_kernel,
        grid=(n // _BM,),
        in_specs=[
            pl.BlockSpec((n, d_in), lambda i: (0, 0)),
            pl.BlockSpec((d_in, d_out), lambda i: (0, 0)),
            pl.BlockSpec((1, d_out), lambda i: (0, 0)),
            pl.BlockSpec((_BM, q), lambda i: (i, 0)),
            pl.BlockSpec((_BM, q), lambda i: (i, 1)),
            pl.BlockSpec(memory_space=pltpu.MemorySpace.HBM),
        ],
        out_specs=pl.BlockSpec((_BM, d_out), lambda i: (i, 0)),
        out_shape=jax.ShapeDtypeStruct((n, d_out), jnp.float32),
        scratch_shapes=[
            pltpu.VMEM((n, d_out), jnp.bfloat16),
            pltpu.VMEM((2, _BM, q), jnp.float32),
            pltpu.SemaphoreType.DMA((2,)),
            pltpu.VMEM((2, _BM, q), jnp.float32),
            pltpu.SemaphoreType.DMA((2,)),
        ],
        compiler_params=pltpu.CompilerParams(
            dimension_semantics=("arbitrary",),
        ),
    )(X, W, b2, A, A, A)
    return out
